# baseline (speedup 1.0000x reference)
import functools

import jax
import jax.numpy as jnp
from jax import lax
from jax.experimental import pallas as pl
from jax.experimental.pallas import tpu as pltpu
from jax.experimental.pallas import tpu_sc as plsc

_FEATURE_DIM = 64
_BOARD_SIZE = 15
_PCODE_DIM = 2380
_EMBED_DIM = 2 * (_PCODE_DIM + 1)
_CELL_DIM = _BOARD_SIZE * _BOARD_SIZE
_CPAD = 240
_LISTCAP = 240
_CHUNK = 16
_SMPAD = 4800


def _sc_embed(s, bd, offs, w_small, w_big, batch):
    info = plsc.get_sparse_core_info()
    nc, ns = info.num_cores, info.num_subcores
    nw = nc * ns
    bpw = batch // nw
    n_f = _FEATURE_DIM * _CELL_DIM

    mesh = plsc.VectorSubcoreMesh(core_axis_name="c", subcore_axis_name="s")

    @functools.partial(
        pl.kernel,
        mesh=mesh,
        out_type=jax.ShapeDtypeStruct((batch * n_f,), jnp.float32),
        compiler_params=pltpu.CompilerParams(
            use_tc_tiling_on_sc=False, needs_layout_passes=False),
        scratch_types=[
            pltpu.VMEM((2, _CPAD), jnp.int32),
            pltpu.VMEM((2, _CPAD), jnp.int32),
            pltpu.VMEM((_CPAD,), jnp.int32),
            pltpu.VMEM((_LISTCAP,), jnp.int32),
            pltpu.VMEM((_LISTCAP,), jnp.int32),
            pltpu.VMEM((_LISTCAP,), jnp.int32),
            pltpu.VMEM((_LISTCAP,), jnp.int32),
            pltpu.VMEM((_LISTCAP,), jnp.int32),
            pltpu.VMEM((_LISTCAP, _FEATURE_DIM), jnp.float32),
            pltpu.VMEM((_LISTCAP, _FEATURE_DIM), jnp.float32),
            pltpu.VMEM((_LISTCAP, _FEATURE_DIM), jnp.float32),
            pltpu.VMEM((_LISTCAP, _FEATURE_DIM), jnp.float32),
            pltpu.VMEM((n_f,), jnp.float32),
            pltpu.VMEM((n_f,), jnp.float32),
            pltpu.VMEM((_CELL_DIM, _FEATURE_DIM), jnp.float32),
            pltpu.VMEM((16,), jnp.int32),
            pltpu.VMEM((16,), jnp.int32),
            pltpu.VMEM((2, _FEATURE_DIM), jnp.float32),
            pltpu.VMEM((2, _FEATURE_DIM), jnp.float32),
            pltpu.VMEM((_FEATURE_DIM,), jnp.float32),
            pltpu.VMEM_SHARED((_CELL_DIM, _FEATURE_DIM), jnp.float32),
            pltpu.SMEM((1,), jnp.int32),
            pltpu.SemaphoreType.DMA,
        ],
    )
    def k(s_hbm, bd_hbm, offs_hbm, wsm_hbm, wbg_hbm, out_hbm,
          s_v, bd_v, offs_v, l0, l1, l2, l3, cid, g0, g1, g2, g3,
          out_v, const_t, const_v, idx2s, idx2b, rows2s, rows2b, crow,
          const_sp, n_ref, sem):
        cidx = lax.axis_index("c")
        sid = lax.axis_index("s")
        wid = sid * nc + cidx
        pltpu.sync_copy(offs_hbm, offs_v)
        zero16 = jnp.zeros((16,), jnp.int32)
        for lst in (l0, l1, l2, l3):
            for q in range(_LISTCAP // 16):
                lst[pl.ds(q * 16, 16)] = zero16
        iota16 = lax.broadcasted_iota(jnp.int32, (16,), 0)
        iota225 = iota16 * _CELL_DIM


        even = (iota16 % 2) == 0
        idx2s[...] = jnp.where(even, jnp.full((16,), _PCODE_DIM, jnp.int32),
                               jnp.full((16,), 2 * _PCODE_DIM + 1, jnp.int32))
        pltpu.async_copy(wsm_hbm.at[idx2s.at[pl.ds(0, 2)]], rows2s,
                         sem).wait()

        def const_body(i, carry):
            c = sid * 15 + i

            @pl.when(c < _CELL_DIM)
            def _():
                off = offs_v[pl.ds(c, 16)][0]
                idx2b[...] = jnp.where(
                    even, off + _PCODE_DIM, off + 2 * _PCODE_DIM + 1)
                pltpu.async_copy(wbg_hbm.at[idx2b.at[pl.ds(0, 2)]], rows2b,
                                 sem).wait()
                for kk in range(_FEATURE_DIM // 16):
                    fs = pl.ds(kk * 16, 16)
                    crow[fs] = (rows2s[0, fs] + rows2s[1, fs]
                                + rows2b[0, fs] + rows2b[1, fs])
                pltpu.sync_copy(crow, const_sp.at[c])
            return carry

        lax.fori_loop(0, 15, const_body, 0)
        plsc.subcore_barrier()
        pltpu.sync_copy(const_sp, const_v)

        def tr_body(c, carry):
            for kk in range(_FEATURE_DIM // 16):
                v = const_v[c, pl.ds(kk * 16, 16)]
                plsc.store_scatter(
                    const_t, [iota225 + (kk * 16 * _CELL_DIM + c)], v)
            return carry

        lax.fori_loop(0, _CELL_DIM, tr_body, 0)

        def batch_body(i, carry):
            b = wid * bpw + i
            pltpu.sync_copy(s_hbm.at[b], s_v)
            pltpu.sync_copy(bd_hbm.at[b], bd_v)

            def compact_body(q, nvec):
                sl = pl.ds(q * 16, 16)
                empty = (bd_v[0, sl] + bd_v[1, sl]) == 0
                cum = plsc.cumsum(jnp.where(empty, 1, 0).astype(jnp.int32))
                cnt = jnp.max(cum)
                pos = nvec + cum - 1
                p0 = s_v[0, sl]
                p1 = s_v[1, sl] + (_PCODE_DIM + 1)
                off = offs_v[sl]
                plsc.store_scatter(l0, [pos], p0, mask=empty)
                plsc.store_scatter(l1, [pos], p1, mask=empty)
                plsc.store_scatter(l2, [pos], p0 + off, mask=empty)
                plsc.store_scatter(l3, [pos], p1 + off, mask=empty)
                plsc.store_scatter(cid, [pos], iota16 + q * 16, mask=empty)
                return nvec + cnt

            nvec = lax.fori_loop(0, _CPAD // 16, compact_body,
                                 jnp.zeros((16,), jnp.int32))
            n = jnp.max(nvec)
            nch = (n + (_CHUNK - 1)) // _CHUNK

            def issue_body(kc, carry2):
                o = pl.multiple_of(kc * _CHUNK, _CHUNK)
                for lst, g, w in ((l0, g0, wsm_hbm), (l1, g1, wsm_hbm),
                                  (l2, g2, wbg_hbm), (l3, g3, wbg_hbm)):
                    pltpu.async_copy(
                        w.at[lst.at[pl.ds(o, _CHUNK)]],
                        g.at[pl.ds(o, _CHUNK)], sem)
                return carry2

            lax.fori_loop(0, nch, issue_body, 0)

            def copy_body(q, carry2):
                for u in range(4):
                    sl = pl.ds(q * 64 + u * 16, 16)
                    out_v[sl] = const_t[sl]
                return carry2

            lax.fori_loop(0, n_f // 64, copy_body, 0)

            def drain_body(kc, carry2):
                o = pl.multiple_of(kc * _CHUNK, _CHUNK)
                for lst, g, w in ((l0, g0, wsm_hbm), (l1, g1, wsm_hbm),
                                  (l2, g2, wbg_hbm), (l3, g3, wbg_hbm)):
                    pltpu.make_async_copy(
                        w.at[lst.at[pl.ds(o, _CHUNK)]],
                        g.at[pl.ds(o, _CHUNK)], sem).wait()
                return carry2

            lax.fori_loop(0, nch, drain_body, 0)

            def fix_body(j, carry2):
                c = cid[pl.ds(j, 16)][0]
                for kk in range(_FEATURE_DIM // 16):
                    fs = pl.ds(kk * 16, 16)
                    v = g0[j, fs] + g1[j, fs] + g2[j, fs] + g3[j, fs]
                    plsc.store_scatter(
                        out_v, [iota225 + (kk * 16 * _CELL_DIM + c)], v)
                return carry2

            lax.fori_loop(0, n, fix_body, 0)
            pltpu.sync_copy(out_v, out_hbm.at[pl.ds(b * n_f, n_f)])
            return carry

        lax.fori_loop(0, bpw, batch_body, 0)

    return k(s, bd, offs, w_small, w_big)


def kernel(sparse_feature_input, sparse_feature_dim, board_input,
           pcode_embedding_W, pcode_board_embedding_W, board_offset):
    del sparse_feature_dim
    batch = sparse_feature_input.shape[0]
    pad = _CPAD - _CELL_DIM
    s = sparse_feature_input[:, 10:12].reshape(batch, 2, _CELL_DIM)
    s = jnp.pad(s, ((0, 0), (0, 0), (0, pad)))
    bd = board_input.reshape(batch, 2, _CELL_DIM)
    bd = jnp.pad(bd, ((0, 0), (0, 0), (0, pad)), constant_values=1)
    offs = jnp.pad(board_offset.reshape(_CELL_DIM), ((0, pad),))
    out = _sc_embed(s, bd, offs, pcode_embedding_W, pcode_board_embedding_W,
                    batch)
    return out.reshape(batch, _FEATURE_DIM, _BOARD_SIZE, _BOARD_SIZE)


# speedup vs baseline: 1.5392x; 1.5392x over previous
import functools

import jax
import jax.numpy as jnp
from jax import lax
from jax.experimental import pallas as pl
from jax.experimental.pallas import tpu as pltpu
from jax.experimental.pallas import tpu_sc as plsc

_FEATURE_DIM = 64
_BOARD_SIZE = 15
_PCODE_DIM = 2380
_EMBED_DIM = 2 * (_PCODE_DIM + 1)
_CELL_DIM = _BOARD_SIZE * _BOARD_SIZE
_CPAD = 240
_LISTCAP = 240
_CHUNK = 16
_SMPAD = 4800


def _sc_embed(sb, w_small, w_big, batch):
    info = plsc.get_sparse_core_info()
    nc, ns = info.num_cores, info.num_subcores
    nw = nc * ns
    bpw = batch // nw
    n_f = _FEATURE_DIM * _CELL_DIM

    mesh = plsc.VectorSubcoreMesh(core_axis_name="c", subcore_axis_name="s")

    @functools.partial(
        pl.kernel,
        mesh=mesh,
        out_type=jax.ShapeDtypeStruct((batch, n_f), jnp.float32),
        compiler_params=pltpu.CompilerParams(
            use_tc_tiling_on_sc=False, needs_layout_passes=False),
        scratch_types=[
            pltpu.VMEM((4, _CELL_DIM), jnp.int32),
            pltpu.VMEM((_LISTCAP,), jnp.int32),
            pltpu.VMEM((_LISTCAP,), jnp.int32),
            pltpu.VMEM((_LISTCAP,), jnp.int32),
            pltpu.VMEM((_LISTCAP,), jnp.int32),
            pltpu.VMEM((_LISTCAP,), jnp.int32),
            pltpu.VMEM((_LISTCAP, _FEATURE_DIM), jnp.float32),
            pltpu.VMEM((_LISTCAP, _FEATURE_DIM), jnp.float32),
            pltpu.VMEM((_LISTCAP, _FEATURE_DIM), jnp.float32),
            pltpu.VMEM((_LISTCAP, _FEATURE_DIM), jnp.float32),
            pltpu.VMEM((n_f,), jnp.float32),
            pltpu.VMEM((n_f,), jnp.float32),
            pltpu.VMEM((_CELL_DIM, _FEATURE_DIM), jnp.float32),
            pltpu.VMEM((16,), jnp.int32),
            pltpu.VMEM((16,), jnp.int32),
            pltpu.VMEM((2, _FEATURE_DIM), jnp.float32),
            pltpu.VMEM((2, _FEATURE_DIM), jnp.float32),
            pltpu.VMEM((_FEATURE_DIM,), jnp.float32),
            pltpu.VMEM_SHARED((_CELL_DIM, _FEATURE_DIM), jnp.float32),
            pltpu.SMEM((1,), jnp.int32),
            pltpu.SemaphoreType.DMA,
        ],
    )
    def k(sb_hbm, wsm_hbm, wbg_hbm, out_hbm,
          sb_v, l0, l1, l2, l3, cid, g0, g1, g2, g3,
          out_v, const_t, const_v, idx2s, idx2b, rows2s, rows2b, crow,
          const_sp, n_ref, sem):
        cidx = lax.axis_index("c")
        sid = lax.axis_index("s")
        wid = sid * nc + cidx
        zero16 = jnp.zeros((16,), jnp.int32)
        for lst in (l0, l1, l2, l3):
            for q in range(_LISTCAP // 16):
                lst[pl.ds(q * 16, 16)] = zero16
        iota16 = lax.broadcasted_iota(jnp.int32, (16,), 0)
        iota225 = iota16 * _CELL_DIM


        even = (iota16 % 2) == 0
        idx2s[...] = jnp.where(even, jnp.full((16,), _PCODE_DIM, jnp.int32),
                               jnp.full((16,), 2 * _PCODE_DIM + 1, jnp.int32))
        pltpu.async_copy(wsm_hbm.at[idx2s.at[pl.ds(0, 2)]], rows2s,
                         sem).wait()

        def const_body(i, carry):
            c = sid * 15 + i

            @pl.when(c < _CELL_DIM)
            def _():
                off = c * _EMBED_DIM
                idx2b[...] = jnp.where(
                    even, off + _PCODE_DIM, off + 2 * _PCODE_DIM + 1)
                pltpu.async_copy(wbg_hbm.at[idx2b.at[pl.ds(0, 2)]], rows2b,
                                 sem).wait()
                for kk in range(_FEATURE_DIM // 16):
                    fs = pl.ds(kk * 16, 16)
                    crow[fs] = (rows2s[0, fs] + rows2s[1, fs]
                                + rows2b[0, fs] + rows2b[1, fs])
                pltpu.sync_copy(crow, const_sp.at[c])
            return carry

        lax.fori_loop(0, 15, const_body, 0)
        plsc.subcore_barrier()
        pltpu.sync_copy(const_sp, const_v)

        def tr_body(c, carry):
            for kk in range(_FEATURE_DIM // 16):
                v = const_v[c, pl.ds(kk * 16, 16)]
                plsc.store_scatter(
                    const_t, [iota225 + (kk * 16 * _CELL_DIM + c)], v)
            return carry

        lax.fori_loop(0, _CELL_DIM, tr_body, 0)

        def batch_body(i, carry):
            b = wid * bpw + i
            pltpu.sync_copy(sb_hbm.at[b], sb_v)

            def compact_chunk(base, lanemask, nvec):
                sl = pl.ds(base, 16)
                empty = (sb_v[2, sl] + sb_v[3, sl]) == 0
                if lanemask is not None:
                    empty = empty & lanemask
                cum = plsc.cumsum(jnp.where(empty, 1, 0).astype(jnp.int32))
                cnt = jnp.max(cum)
                pos = nvec + cum - 1
                p0 = sb_v[0, sl]
                p1 = sb_v[1, sl] + (_PCODE_DIM + 1)
                cells = iota16 + base
                off = cells * _EMBED_DIM
                plsc.store_scatter(l0, [pos], p0, mask=empty)
                plsc.store_scatter(l1, [pos], p1, mask=empty)
                plsc.store_scatter(l2, [pos], p0 + off, mask=empty)
                plsc.store_scatter(l3, [pos], p1 + off, mask=empty)
                plsc.store_scatter(cid, [pos], cells, mask=empty)
                return nvec + cnt

            def compact_body(q, nvec):
                return compact_chunk(q * 16, None, nvec)

            nvec = lax.fori_loop(0, 14, compact_body,
                                 jnp.zeros((16,), jnp.int32))
            nvec = compact_chunk(209, iota16 == 15, nvec)
            n = jnp.max(nvec)
            nch = (n + (_CHUNK - 1)) // _CHUNK

            def issue_body(kc, carry2):
                o = pl.multiple_of(kc * _CHUNK, _CHUNK)
                for lst, g, w in ((l0, g0, wsm_hbm), (l1, g1, wsm_hbm),
                                  (l2, g2, wbg_hbm), (l3, g3, wbg_hbm)):
                    pltpu.async_copy(
                        w.at[lst.at[pl.ds(o, _CHUNK)]],
                        g.at[pl.ds(o, _CHUNK)], sem)
                return carry2

            lax.fori_loop(0, nch, issue_body, 0)

            def copy_body(q, carry2):
                for u in range(4):
                    sl = pl.ds(q * 64 + u * 16, 16)
                    out_v[sl] = const_t[sl]
                return carry2

            lax.fori_loop(0, n_f // 64, copy_body, 0)

            def drain_body(kc, carry2):
                o = pl.multiple_of(kc * _CHUNK, _CHUNK)
                for lst, g, w in ((l0, g0, wsm_hbm), (l1, g1, wsm_hbm),
                                  (l2, g2, wbg_hbm), (l3, g3, wbg_hbm)):
                    pltpu.make_async_copy(
                        w.at[lst.at[pl.ds(o, _CHUNK)]],
                        g.at[pl.ds(o, _CHUNK)], sem).wait()
                return carry2

            lax.fori_loop(0, nch, drain_body, 0)

            def fix_body(j, carry2):
                c = cid[pl.ds(j, 16)][0]
                for kk in range(_FEATURE_DIM // 16):
                    fs = pl.ds(kk * 16, 16)
                    v = g0[j, fs] + g1[j, fs] + g2[j, fs] + g3[j, fs]
                    plsc.store_scatter(
                        out_v, [iota225 + (kk * 16 * _CELL_DIM + c)], v)
                return carry2

            lax.fori_loop(0, n, fix_body, 0)
            pltpu.sync_copy(out_v, out_hbm.at[b])
            return carry

        lax.fori_loop(0, bpw, batch_body, 0)

    return k(sb, w_small, w_big)


def kernel(sparse_feature_input, sparse_feature_dim, board_input,
           pcode_embedding_W, pcode_board_embedding_W, board_offset):
    del sparse_feature_dim, board_offset
    batch = sparse_feature_input.shape[0]
    sb = jnp.concatenate(
        [sparse_feature_input[:, 10:12].reshape(batch, 2, _CELL_DIM),
         board_input.reshape(batch, 2, _CELL_DIM)], axis=1)
    out = _sc_embed(sb, pcode_embedding_W, pcode_board_embedding_W, batch)
    return out.reshape(batch, _FEATURE_DIM, _BOARD_SIZE, _BOARD_SIZE)


# speedup vs baseline: 1.5843x; 1.0293x over previous
import functools

import jax
import jax.numpy as jnp
from jax import lax
from jax.experimental import pallas as pl
from jax.experimental.pallas import tpu as pltpu
from jax.experimental.pallas import tpu_sc as plsc

_FEATURE_DIM = 64
_BOARD_SIZE = 15
_PCODE_DIM = 2380
_EMBED_DIM = 2 * (_PCODE_DIM + 1)
_CELL_DIM = _BOARD_SIZE * _BOARD_SIZE
_LISTCAP = 240
_CHUNK = 8


def _sc_embed(sb, w_small, w_big, batch):
    info = plsc.get_sparse_core_info()
    nc, ns = info.num_cores, info.num_subcores
    nw = nc * ns
    bpw = batch // nw
    n_f = _FEATURE_DIM * _CELL_DIM

    mesh = plsc.VectorSubcoreMesh(core_axis_name="c", subcore_axis_name="s")

    @functools.partial(
        pl.kernel,
        mesh=mesh,
        out_type=jax.ShapeDtypeStruct((batch, n_f), jnp.float32),
        compiler_params=pltpu.CompilerParams(
            use_tc_tiling_on_sc=False, needs_layout_passes=False),
        scratch_types=[
            pltpu.VMEM((4 * (_CELL_DIM + 1),), jnp.int32),
            pltpu.VMEM((_LISTCAP,), jnp.int32),
            pltpu.VMEM((_LISTCAP,), jnp.int32),
            pltpu.VMEM((_LISTCAP,), jnp.int32),
            pltpu.VMEM((_LISTCAP,), jnp.int32),
            pltpu.VMEM((_LISTCAP,), jnp.int32),
            pltpu.VMEM((_LISTCAP, _FEATURE_DIM), jnp.float32),
            pltpu.VMEM((_LISTCAP, _FEATURE_DIM), jnp.float32),
            pltpu.VMEM((_LISTCAP, _FEATURE_DIM), jnp.float32),
            pltpu.VMEM((_LISTCAP, _FEATURE_DIM), jnp.float32),
            pltpu.VMEM((n_f,), jnp.float32),
            pltpu.VMEM((n_f,), jnp.float32),
            pltpu.VMEM((_CELL_DIM, _FEATURE_DIM), jnp.float32),
            pltpu.VMEM((16,), jnp.int32),
            pltpu.VMEM((16,), jnp.int32),
            pltpu.VMEM((2, _FEATURE_DIM), jnp.float32),
            pltpu.VMEM((2, _FEATURE_DIM), jnp.float32),
            pltpu.VMEM((_FEATURE_DIM,), jnp.float32),
            pltpu.VMEM_SHARED((_CELL_DIM, _FEATURE_DIM), jnp.float32),
            pltpu.SemaphoreType.DMA,
        ],
    )
    def k(sb_hbm, wsm_hbm, wbg_hbm, out_hbm,
          sb_v, l0, l1, l2, l3, cid, g0, g1, g2, g3,
          out_v, const_t, const_v, idx2s, idx2b, rows2s, rows2b, crow,
          const_sp, sem):
        cidx = lax.axis_index("c")
        sid = lax.axis_index("s")
        wid = sid * nc + cidx
        zero16 = jnp.zeros((16,), jnp.int32)
        for lst in (l0, l1, l2, l3):
            for q in range(_LISTCAP // 16):
                lst[pl.ds(q * 16, 16)] = zero16
        iota16 = lax.broadcasted_iota(jnp.int32, (16,), 0)
        iota225 = iota16 * _CELL_DIM


        even = (iota16 % 2) == 0
        idx2s[...] = jnp.where(even, jnp.full((16,), _PCODE_DIM, jnp.int32),
                               jnp.full((16,), 2 * _PCODE_DIM + 1, jnp.int32))
        pltpu.async_copy(wsm_hbm.at[idx2s.at[pl.ds(0, 2)]], rows2s,
                         sem).wait()

        def const_body(i, carry):
            c = sid * 15 + i

            @pl.when(c < _CELL_DIM)
            def _():
                off = c * _EMBED_DIM
                idx2b[...] = jnp.where(
                    even, off + _PCODE_DIM, off + 2 * _PCODE_DIM + 1)
                pltpu.async_copy(wbg_hbm.at[idx2b.at[pl.ds(0, 2)]], rows2b,
                                 sem).wait()
                for kk in range(_FEATURE_DIM // 16):
                    fs = pl.ds(kk * 16, 16)
                    crow[fs] = (rows2s[0, fs] + rows2s[1, fs]
                                + rows2b[0, fs] + rows2b[1, fs])
                pltpu.sync_copy(crow, const_sp.at[c])
            return carry

        lax.fori_loop(0, 15, const_body, 0)
        plsc.subcore_barrier()
        pltpu.sync_copy(const_sp, const_v)

        def tr_body(c, carry):
            for kk in range(_FEATURE_DIM // 16):
                v = const_v[c, pl.ds(kk * 16, 16)]
                plsc.store_scatter(
                    const_t, [iota225 + (kk * 16 * _CELL_DIM + c)], v)
            return carry

        lax.fori_loop(0, _CELL_DIM, tr_body, 0)

        blk = 4 * (_CELL_DIM + 1)

        def batch_body(i, carry):
            b = wid * bpw + i
            pltpu.sync_copy(sb_hbm.at[pl.ds(b * blk, blk)], sb_v)

            def compact_chunk(base, lanemask, nvec):
                row = _CELL_DIM + 1
                empty = (sb_v[pl.ds(2 * row + base, 16)]
                         + sb_v[pl.ds(3 * row + base, 16)]) == 0
                if lanemask is not None:
                    empty = empty & lanemask
                cum = plsc.cumsum(jnp.where(empty, 1, 0).astype(jnp.int32))
                cnt = jnp.max(cum)
                pos = nvec + cum - 1
                p0 = sb_v[pl.ds(base, 16)]
                p1 = sb_v[pl.ds(row + base, 16)] + (_PCODE_DIM + 1)
                cells = iota16 + base
                off = cells * _EMBED_DIM
                plsc.store_scatter(l0, [pos], p0, mask=empty)
                plsc.store_scatter(l1, [pos], p1, mask=empty)
                plsc.store_scatter(l2, [pos], p0 + off, mask=empty)
                plsc.store_scatter(l3, [pos], p1 + off, mask=empty)
                plsc.store_scatter(cid, [pos], cells, mask=empty)
                return nvec + cnt

            def compact_body(q, nvec):
                return compact_chunk(q * 16, None, nvec)

            nvec = lax.fori_loop(0, 14, compact_body,
                                 jnp.zeros((16,), jnp.int32))
            nvec = compact_chunk(209, iota16 == 15, nvec)
            n = jnp.max(nvec)
            nch = (n + (_CHUNK - 1)) // _CHUNK

            def issue_body(kc, carry2):
                o = pl.multiple_of(kc * _CHUNK, _CHUNK)
                for lst, g, w in ((l0, g0, wsm_hbm), (l1, g1, wsm_hbm),
                                  (l2, g2, wbg_hbm), (l3, g3, wbg_hbm)):
                    pltpu.async_copy(
                        w.at[lst.at[pl.ds(o, _CHUNK)]],
                        g.at[pl.ds(o, _CHUNK)], sem)
                return carry2

            lax.fori_loop(0, nch, issue_body, 0)

            def copy_body(q, carry2):
                for u in range(4):
                    sl = pl.ds(q * 64 + u * 16, 16)
                    out_v[sl] = const_t[sl]
                return carry2

            lax.fori_loop(0, n_f // 64, copy_body, 0)

            def drain_body(kc, carry2):
                o = pl.multiple_of(kc * _CHUNK, _CHUNK)
                for lst, g, w in ((l0, g0, wsm_hbm), (l1, g1, wsm_hbm),
                                  (l2, g2, wbg_hbm), (l3, g3, wbg_hbm)):
                    pltpu.make_async_copy(
                        w.at[lst.at[pl.ds(o, _CHUNK)]],
                        g.at[pl.ds(o, _CHUNK)], sem).wait()
                return carry2

            lax.fori_loop(0, nch, drain_body, 0)

            def fix_body(j, carry2):
                c = cid[pl.ds(j, 16)][0]
                for kk in range(_FEATURE_DIM // 16):
                    fs = pl.ds(kk * 16, 16)
                    v = g0[j, fs] + g1[j, fs] + g2[j, fs] + g3[j, fs]
                    plsc.store_scatter(
                        out_v, [iota225 + (kk * 16 * _CELL_DIM + c)], v)
                return carry2

            lax.fori_loop(0, n, fix_body, 0)
            pltpu.sync_copy(out_v, out_hbm.at[b])
            return carry

        lax.fori_loop(0, bpw, batch_body, 0)

    return k(sb, w_small, w_big)


def kernel(sparse_feature_input, sparse_feature_dim, board_input,
           pcode_embedding_W, pcode_board_embedding_W, board_offset):
    del sparse_feature_dim, board_offset
    batch = sparse_feature_input.shape[0]
    sb = jnp.concatenate(
        [sparse_feature_input[:, 10:12].reshape(batch, 2, _CELL_DIM),
         board_input.reshape(batch, 2, _CELL_DIM)], axis=1)
    sb = jnp.pad(sb, ((0, 0), (0, 0), (0, 1))).reshape(batch * 4 *
                                                       (_CELL_DIM + 1))
    out = _sc_embed(sb, pcode_embedding_W, pcode_board_embedding_W, batch)
    return out.reshape(batch, _FEATURE_DIM, _BOARD_SIZE, _BOARD_SIZE)
